# baseline (speedup 1.0000x reference)
import functools

import jax
import jax.numpy as jnp
from jax import lax
from jax.experimental import pallas as pl
from jax.experimental.pallas import tpu as pltpu
from jax.experimental.pallas import tpu_sc as plsc

B, L, D = 16, 4096, 512
LANES = 16
NCORES, NSUB = 2, 16
NW = NCORES * NSUB
CHUNK = 16
NCH = L // CHUNK
IPR = NCH // NW
VPP = D // LANES


def _scan_body(x_hbm, out_hbm, buf, res_v, sem0, sem1, semw):
    c = lax.axis_index("c")
    s = lax.axis_index("s")
    w = s * NCORES + c

    zeros = jnp.zeros((LANES,), jnp.float32)
    lane = lax.iota(jnp.int32, LANES)
    sems = (sem0, sem1)

    def vmax_scalar(v):
        for k_ in (1, 2, 4, 8):
            v = jnp.maximum(v, jnp.take(v, lane ^ k_))
        return v[0]

    def absmax(base):
        def g_body(p, accs):
            accs = list(accs)
            for i in range(VPP):
                accs[i % 4] = jnp.maximum(
                    accs[i % 4], jnp.abs(buf[base + p, pl.ds(i * LANES, LANES)]))
            return tuple(accs)
        a0, a1, a2, a3 = lax.fori_loop(
            0, CHUNK, g_body, (zeros, zeros, zeros, zeros))
        return jnp.maximum(jnp.maximum(a0, a1), jnp.maximum(a2, a3))

    def copy(r, jc, par, sem):
        return pltpu.make_async_copy(
            x_hbm.at[r, pl.ds(jc * CHUNK, CHUNK), :],
            buf.at[pl.ds(par * CHUNK, CHUNK), :], sem)

    def scan_row(r):
        jc0 = NCH - 1 - w
        copy(r, jc0, 0, sems[0]).start()

        def step(i, par, fj):
            def do(_):
                jc = jc0 - NW * i
                jcn = jnp.maximum(jc - NW, 0)
                copy(r, jcn, 1 - par, sems[1 - par]).start()
                copy(r, jc, par, sems[par]).wait()
                found = vmax_scalar(absmax(par * CHUNK)) > 0.0
                return jnp.where(found, jc, -1)

            return lax.cond(fj < 0, do, lambda _: fj, 0)

        def iter2_body(it2, st):
            fj, cnt = st
            for p_ in (0, 1):
                i = 2 * it2 + p_
                nfj = step(i, p_, fj)
                cnt = jnp.where(fj < 0, cnt + 1, cnt)
                fj = nfj
            return (fj, cnt)

        fj, cnt = lax.fori_loop(0, IPR // 2, iter2_body,
                                (jnp.int32(-1), jnp.int32(0)))

        def drain(par):
            copy(r, 0, par, sems[par]).wait()
            return 0

        _ = lax.cond(cnt % 2 == 0, lambda _: drain(0), lambda _: drain(1), 0)

        def resolve(fj_):
            base = ((cnt - 1) % 2) * CHUNK

            def p_body(p, last_p):
                acc = zeros
                for i in range(VPP):
                    acc = jnp.maximum(
                        acc, jnp.abs(buf[base + p, pl.ds(i * LANES, LANES)]))
                nz = vmax_scalar(acc) > 0.0
                return jnp.where(nz, p, last_p)

            last_p = lax.fori_loop(0, CHUNK, p_body, jnp.int32(0))
            return fj_ * CHUNK + last_p + 1

        return lax.cond(fj >= 0, resolve, lambda _: jnp.int32(0), fj)

    def q_body(q, res):
        r = (s + q) % B
        best = scan_row(r)
        return jnp.maximum(res, jnp.where(lane == r, best, 0))

    res_v[:] = lax.fori_loop(0, B, q_body, jnp.zeros((LANES,), jnp.int32))
    cp = pltpu.make_async_copy(res_v, out_hbm.at[w], semw)
    cp.start()
    cp.wait()


_scan_kernel = functools.partial(
    pl.kernel,
    out_type=jax.ShapeDtypeStruct((NW, NSUB), jnp.int32),
    mesh=plsc.VectorSubcoreMesh(core_axis_name="c", subcore_axis_name="s"),
    scratch_types=[
        pltpu.VMEM((2 * CHUNK, D), jnp.float32),
        pltpu.VMEM((LANES,), jnp.int32),
        pltpu.SemaphoreType.DMA,
        pltpu.SemaphoreType.DMA,
        pltpu.SemaphoreType.DMA,
    ],
)(_scan_body)


def _copy_body(x_ref, o_ref):
    o_ref[...] = x_ref[...]


_BS = 2048

_values_copy = pl.pallas_call(
    _copy_body,
    grid=(B * L // _BS,),
    in_specs=[pl.BlockSpec((_BS, D), lambda i: (i, 0))],
    out_specs=pl.BlockSpec((_BS, D), lambda i: (i, 0)),
    out_shape=jax.ShapeDtypeStruct((B * L, D), jnp.float32),
)


def kernel(inputs):
    x3 = inputs.reshape(B, L, D)
    cand = _scan_kernel(x3)
    values = _values_copy(inputs.reshape(B * L, D)).reshape(B, L, D)
    row_lengths = jnp.max(cand, axis=0).astype(jnp.int32)
    return (values, row_lengths)


# speedup vs baseline: 1.3072x; 1.3072x over previous
import functools

import jax
import jax.numpy as jnp
from jax import lax
from jax.experimental import pallas as pl
from jax.experimental.pallas import tpu as pltpu
from jax.experimental.pallas import tpu_sc as plsc

B, L, D = 16, 4096, 512
LANES = 16
NCORES, NSUB = 2, 16
NW = NCORES * NSUB
HALF = L // 2
CHUNK = 64
NCH = HALF // CHUNK
NB = 3
VPP = D // LANES


def _body(x_hbm, values_hbm, cand_hbm, buf, res_v, rsems, wsems, semw):
    c = lax.axis_index("c")
    s = lax.axis_index("s")
    w = s * NCORES + c
    b = w // 2
    h = w % 2
    base_pos = h * HALF

    zeros = jnp.zeros((LANES,), jnp.float32)
    lane = lax.iota(jnp.int32, LANES)

    def vmax_scalar(v):
        for k_ in (1, 2, 4, 8):
            v = jnp.maximum(v, jnp.take(v, lane ^ k_))
        return v[0]

    def rd(k, q):
        return pltpu.make_async_copy(
            x_hbm.at[b, pl.ds(base_pos + k * CHUNK, CHUNK), :],
            buf.at[pl.ds(q * CHUNK, CHUNK), :], rsems[q])

    def wr(k, q):
        return pltpu.make_async_copy(
            buf.at[pl.ds(q * CHUNK, CHUNK), :],
            values_hbm.at[b, pl.ds(base_pos + k * CHUNK, CHUNK), :], wsems[q])

    def absmax(q):
        def g_body(p, accs):
            accs = list(accs)
            for i in range(VPP):
                accs[i % 4] = jnp.maximum(
                    accs[i % 4],
                    jnp.abs(buf[q * CHUNK + p, pl.ds(i * LANES, LANES)]))
            return tuple(accs)
        a0, a1, a2, a3 = lax.fori_loop(
            0, CHUNK, g_body, (zeros, zeros, zeros, zeros))
        return jnp.maximum(jnp.maximum(a0, a1), jnp.maximum(a2, a3))

    def step(k, p, best, first):
        if not first:
            wr(0, (p - 1) % NB).wait()
        kr = jnp.minimum(k + (NB - 1), NCH - 1)
        rd(kr, (p + NB - 1) % NB).start()
        rd(k, p).wait()
        found = vmax_scalar(absmax(p)) > 0.0
        best = jnp.where(found, k, best)
        wr(k, p).start()
        return best

    for q in range(NB - 1):
        rd(q, q).start()

    best = jnp.int32(-1)
    for k0 in range(2):
        best = step(jnp.int32(k0), k0 % NB, best, first=(k0 == 0))

    def group(g, best):
        for j in range(NB):
            k = 2 + g * NB + j
            best = step(k, (2 + j) % NB, best, first=False)
        return best

    best = lax.fori_loop(0, (NCH - 2) // NB, group, best)

    wr(0, (NCH - 1) % NB).wait()
    rd(0, 0).wait()
    rd(0, 2).wait()

    def resolve(best_):
        rd(best_, 0).start()
        rd(best_, 0).wait()

        def p_body(p, last_p):
            acc = zeros
            for i in range(VPP):
                acc = jnp.maximum(
                    acc, jnp.abs(buf[p, pl.ds(i * LANES, LANES)]))
            nz = vmax_scalar(acc) > 0.0
            return jnp.where(nz, p, last_p)

        last_p = lax.fori_loop(0, CHUNK, p_body, jnp.int32(0))
        return base_pos + best_ * CHUNK + last_p + 1

    length = lax.cond(best >= 0, resolve, lambda _: jnp.int32(0), best)

    res_v[:] = jnp.where(lane == b, length, 0)
    cp = pltpu.make_async_copy(res_v, cand_hbm.at[w], semw)
    cp.start()
    cp.wait()


_fused_kernel = functools.partial(
    pl.kernel,
    out_type=(
        jax.ShapeDtypeStruct((B, L, D), jnp.float32),
        jax.ShapeDtypeStruct((NW, NSUB), jnp.int32),
    ),
    mesh=plsc.VectorSubcoreMesh(core_axis_name="c", subcore_axis_name="s"),
    scratch_types=[
        pltpu.VMEM((NB * CHUNK, D), jnp.float32),
        pltpu.VMEM((LANES,), jnp.int32),
        [pltpu.SemaphoreType.DMA] * NB,
        [pltpu.SemaphoreType.DMA] * NB,
        pltpu.SemaphoreType.DMA,
    ],
)(_body)


def kernel(inputs):
    values, cand = _fused_kernel(inputs.reshape(B, L, D))
    row_lengths = jnp.max(cand, axis=0).astype(jnp.int32)
    return (values, row_lengths)


# speedup vs baseline: 1.3498x; 1.0326x over previous
import functools

import jax
import jax.numpy as jnp
from jax import lax
from jax.experimental import pallas as pl
from jax.experimental.pallas import tpu as pltpu
from jax.experimental.pallas import tpu_sc as plsc

B, L, D = 16, 4096, 512
LANES = 16
NCORES, NSUB = 2, 16
NW = NCORES * NSUB
HALF = L // 2
CHUNK = 32
NCH = HALF // CHUNK
NB = 4
RA = 2
VPP = D // LANES


def _body(x_hbm, values_hbm, cand_hbm, buf, res_v, rsems, wsems, semw):
    c = lax.axis_index("c")
    s = lax.axis_index("s")
    w = s * NCORES + c
    b = w // 2
    h = w % 2
    base_pos = h * HALF

    zeros = jnp.zeros((LANES,), jnp.float32)
    lane = lax.iota(jnp.int32, LANES)

    def vmax_scalar(v):
        for k_ in (1, 2, 4, 8):
            v = jnp.maximum(v, jnp.take(v, lane ^ k_))
        return v[0]

    def rd(k, q):
        return pltpu.make_async_copy(
            x_hbm.at[b, pl.ds(base_pos + k * CHUNK, CHUNK), :],
            buf.at[pl.ds(q * CHUNK, CHUNK), :], rsems[q])

    def wr(k, q):
        return pltpu.make_async_copy(
            buf.at[pl.ds(q * CHUNK, CHUNK), :],
            values_hbm.at[b, pl.ds(base_pos + k * CHUNK, CHUNK), :], wsems[q])

    def absmax(q):
        def g_body(p, accs):
            accs = list(accs)
            for i in range(VPP):
                accs[i % 4] = jnp.maximum(
                    accs[i % 4],
                    jnp.abs(buf[q * CHUNK + p, pl.ds(i * LANES, LANES)]))
            return tuple(accs)
        a0, a1, a2, a3 = lax.fori_loop(
            0, CHUNK, g_body, (zeros, zeros, zeros, zeros))
        return jnp.maximum(jnp.maximum(a0, a1), jnp.maximum(a2, a3))

    def step(k, p, best, first):
        if not first:
            wr(0, (p - RA) % NB).wait()
        kr = jnp.minimum(k + RA, NCH - 1)
        rd(kr, (p + RA) % NB).start()
        rd(k, p).wait()
        found = vmax_scalar(absmax(p)) > 0.0
        best = jnp.where(found, k, best)
        wr(k, p).start()
        return best

    for q in range(RA):
        rd(q, q).start()

    best = jnp.int32(-1)
    for k0 in range(RA):
        best = step(jnp.int32(k0), k0 % NB, best, first=True)

    def group(g, best):
        for j in range(NB):
            k = RA + g * NB + j
            best = step(k, (RA + j) % NB, best, first=False)
        return best

    best = lax.fori_loop(0, (NCH - 2 * RA) // NB, group, best)

    for k0 in range(NCH - RA, NCH):
        best = step(jnp.int32(k0), k0 % NB, best, first=False)

    wr(0, (NCH - 2) % NB).wait()
    wr(0, (NCH - 1) % NB).wait()
    for q in range(RA):
        rd(0, q).wait()

    def resolve(best_):
        rd(best_, 0).start()
        rd(best_, 0).wait()

        def p_body(p, last_p):
            acc = zeros
            for i in range(VPP):
                acc = jnp.maximum(
                    acc, jnp.abs(buf[p, pl.ds(i * LANES, LANES)]))
            nz = vmax_scalar(acc) > 0.0
            return jnp.where(nz, p, last_p)

        last_p = lax.fori_loop(0, CHUNK, p_body, jnp.int32(0))
        return base_pos + best_ * CHUNK + last_p + 1

    length = lax.cond(best >= 0, resolve, lambda _: jnp.int32(0), best)

    res_v[:] = jnp.where(lane == b, length, 0)
    cp = pltpu.make_async_copy(res_v, cand_hbm.at[w], semw)
    cp.start()
    cp.wait()


_fused_kernel = functools.partial(
    pl.kernel,
    out_type=(
        jax.ShapeDtypeStruct((B, L, D), jnp.float32),
        jax.ShapeDtypeStruct((NW, NSUB), jnp.int32),
    ),
    mesh=plsc.VectorSubcoreMesh(core_axis_name="c", subcore_axis_name="s"),
    scratch_types=[
        pltpu.VMEM((NB * CHUNK, D), jnp.float32),
        pltpu.VMEM((LANES,), jnp.int32),
        [pltpu.SemaphoreType.DMA] * NB,
        [pltpu.SemaphoreType.DMA] * NB,
        pltpu.SemaphoreType.DMA,
    ],
)(_body)


def kernel(inputs):
    values, cand = _fused_kernel(inputs.reshape(B, L, D))
    row_lengths = jnp.max(cand, axis=0).astype(jnp.int32)
    return (values, row_lengths)
